# Initial kernel scaffold; baseline (speedup 1.0000x reference)
import jax
import jax.numpy as jnp
from jax.experimental import pallas as pl


def kernel(x, edge_index, Wl_mu, bl_mu, Wr_mu, Wl_ls, bl_ls, Wr_ls):
    raise NotImplementedError("write your pallas kernel here")


# speedup vs baseline: 1.9644x; 1.9644x over previous
import functools

import jax
import jax.numpy as jnp
from jax import lax
from jax.experimental import pallas as pl
from jax.experimental.pallas import tpu as pltpu
from jax.experimental.pallas import tpu_sc as plsc

N_NODES = 10000
N_EDGES = 160000
D_IN = 256
D_OUT = 256
DH = 128
N_PASSES = 2
NR = 5120
ACC_ROWS = 5632
LOC_DUMMY = NR
GLOB_DUMMY = N_NODES
E_PAD = 163840
ROWS_PER_SUB = 80
ACC_PER_SUB = ACC_ROWS // 16
OUT_PER_SUB = NR // 16
HIST_ROWS = 80


def _sc_aggregate(x2, src2, dst2):
  mesh = plsc.VectorSubcoreMesh(core_axis_name="c", subcore_axis_name="s")

  @functools.partial(
      pl.kernel,
      mesh=mesh,
      compiler_params=pltpu.CompilerParams(needs_layout_passes=False),
      out_type=[
          jax.ShapeDtypeStruct((2, N_PASSES * NR, DH), jnp.float32),
          jax.ShapeDtypeStruct((16, HIST_ROWS, 128), jnp.float32),
      ],
      scratch_types=[
          pltpu.VMEM((ROWS_PER_SUB, 128), jnp.int32),
          pltpu.VMEM((ROWS_PER_SUB, 128), jnp.int32),
          pltpu.VMEM((ROWS_PER_SUB, 128), jnp.int32),
          pltpu.VMEM((128, DH), jnp.float32),
          pltpu.VMEM((HIST_ROWS, 128), jnp.float32),
          pltpu.VMEM((128, DH), jnp.float32),
          pltpu.VMEM_SHARED((ACC_ROWS, DH), jnp.float32),
          pltpu.SemaphoreType.DMA,
      ],
  )
  def agg_kernel(x2_hbm, src_hbm, dst_hbm, out_hbm, cnt_hbm,
                 src_v, dst_v, dstp_v, rows_v, hist_v, z_v, acc_sh, sem):
    c = lax.axis_index("c")
    s = lax.axis_index("s")

    def fill_row(r, _):
      for k in range(DH // 16):
        z_v[r, pl.ds(k * 16, 16)] = jnp.zeros((16,), jnp.float32)
      return 0
    lax.fori_loop(0, 128, fill_row, 0)

    def hist_zero_row(r, _):
      for k in range(128 // 16):
        hist_v[r, pl.ds(k * 16, 16)] = jnp.zeros((16,), jnp.float32)
      return 0
    lax.fori_loop(0, HIST_ROWS, hist_zero_row, 0)

    row0 = s * ROWS_PER_SUB
    pltpu.sync_copy(src_hbm.at[pl.ds(row0, ROWS_PER_SUB)], src_v)
    pltpu.sync_copy(dst_hbm.at[pl.ds(row0, ROWS_PER_SUB)], dst_v)
    off = c * N_NODES

    def src_off_row(r, _):
      for k in range(128 // 16):
        src_v[r, pl.ds(k * 16, 16)] = src_v[r, pl.ds(k * 16, 16)] + off
      return 0
    lax.fori_loop(0, ROWS_PER_SUB, src_off_row, 0)

    @pl.when(c == 0)
    def _histogram():
      ones16 = jnp.ones((16,), jnp.float32)

      def hist_row(r, _):
        for k in range(128 // 16):
          v = dst_v[r, pl.ds(k * 16, 16)]
          plsc.addupdate_scatter(
              hist_v, [lax.shift_right_logical(v, 7), v & 127], ones16)
        return 0
      lax.fori_loop(0, ROWS_PER_SUB, hist_row, 0)
      pltpu.sync_copy(hist_v, cnt_hbm.at[s])

    for p in range(N_PASSES):
      cb = s * ACC_PER_SUB
      pltpu.sync_copy(z_v, acc_sh.at[pl.ds(cb, 128)])
      pltpu.sync_copy(z_v, acc_sh.at[pl.ds(cb + 128, 128)])
      pltpu.sync_copy(z_v.at[pl.ds(0, 96)], acc_sh.at[pl.ds(cb + 256, 96)])

      def remap_row(r, _):
        for k in range(128 // 16):
          v = dst_v[r, pl.ds(k * 16, 16)]
          d = v - (p * NR)
          ok = (d >= 0) & (d < NR)
          dstp_v[r, pl.ds(k * 16, 16)] = jnp.where(
              ok, d, jnp.full((16,), LOC_DUMMY, jnp.int32))
        return 0
      lax.fori_loop(0, ROWS_PER_SUB, remap_row, 0)

      plsc.subcore_barrier()

      def edge_step(j, _):
        pltpu.async_copy(x2_hbm.at[src_v.at[j]], rows_v, sem).wait()
        pltpu.sync_copy(rows_v, acc_sh.at[dstp_v.at[j]], add=True)
        return 0
      lax.fori_loop(0, ROWS_PER_SUB, edge_step, 0)

      plsc.subcore_barrier()

      ob = s * OUT_PER_SUB
      for (o, w) in ((0, 128), (128, 128), (256, 64)):
        pltpu.sync_copy(acc_sh.at[pl.ds(ob + o, w)], rows_v.at[pl.ds(0, w)])
        pltpu.sync_copy(rows_v.at[pl.ds(0, w)],
                        out_hbm.at[c, pl.ds(p * NR + ob + o, w)])

      if p + 1 < N_PASSES:
        plsc.subcore_barrier()

  return agg_kernel(x2, src2, dst2)


def _tc_body(sumL_ref, sumR_ref, cnt_ref, x_ref, w_ref, b_ref, out_ref):
  cnt = jnp.sum(cnt_ref[...], axis=1)[:, None]
  r = 1.0 / jnp.maximum(cnt, 1.0)
  aggL = sumL_ref[...] * r
  aggR = sumR_ref[...] * r
  out_ref[...] = (
      jnp.dot(aggL, w_ref[0:DH, :], preferred_element_type=jnp.float32)
      + jnp.dot(aggR, w_ref[DH:D_IN, :], preferred_element_type=jnp.float32)
      + jnp.dot(x_ref[...], w_ref[D_IN:, :], preferred_element_type=jnp.float32)
      + b_ref[...]
  )


def _tc_matmul(sumL, sumR, cnt, x, w_all, b_all):
  blk = 1000
  grid = (N_NODES // blk,)
  return pl.pallas_call(
      _tc_body,
      grid=grid,
      in_specs=[
          pl.BlockSpec((blk, DH), lambda i: (i, 0)),
          pl.BlockSpec((blk, DH), lambda i: (i, 0)),
          pl.BlockSpec((blk, 16), lambda i: (i, 0)),
          pl.BlockSpec((blk, D_IN), lambda i: (i, 0)),
          pl.BlockSpec((2 * D_IN, 2 * D_OUT), lambda i: (0, 0)),
          pl.BlockSpec((1, 2 * D_OUT), lambda i: (0, 0)),
      ],
      out_specs=pl.BlockSpec((blk, 2 * D_OUT), lambda i: (i, 0)),
      out_shape=jax.ShapeDtypeStruct((N_NODES, 2 * D_OUT), jnp.float32),
  )(sumL, sumR, cnt, x, w_all, b_all)


def kernel(x, edge_index, Wl_mu, bl_mu, Wr_mu, Wl_ls, bl_ls, Wr_ls):
  x2 = jnp.concatenate([x[:, :DH], x[:, DH:]], axis=0)
  src = edge_index[0]
  dst = edge_index[1]
  pad = E_PAD - N_EDGES
  srcp = jnp.concatenate([src, jnp.zeros((pad,), jnp.int32)])
  dstp = jnp.concatenate([dst, jnp.full((pad,), GLOB_DUMMY, jnp.int32)])
  src2 = srcp.reshape(E_PAD // 128, 128)
  dst2 = dstp.reshape(E_PAD // 128, 128)

  summed2, hist = _sc_aggregate(x2, src2, dst2)
  summed2 = summed2[:, :N_NODES, :]
  cnt16 = hist.reshape(16, N_PASSES * NR).T[:N_NODES]

  w_all = jnp.concatenate(
      [jnp.concatenate([Wl_mu.T, Wl_ls.T], axis=1),
       jnp.concatenate([Wr_mu.T, Wr_ls.T], axis=1)], axis=0)
  b_all = jnp.concatenate([bl_mu, bl_ls]).reshape(1, 2 * D_OUT)

  out = _tc_matmul(summed2[0], summed2[1], cnt16, x, w_all, b_all)
  return (out[:, :D_OUT], out[:, D_OUT:])


# speedup vs baseline: 2.2159x; 1.1280x over previous
import functools

import jax
import jax.numpy as jnp
from jax import lax
from jax.experimental import pallas as pl
from jax.experimental.pallas import tpu as pltpu
from jax.experimental.pallas import tpu_sc as plsc

N_NODES = 10000
N_EDGES = 160000
D_IN = 256
D_OUT = 256
DH = 128
N_PASSES = 2
NR = 5120
ACC_ROWS = 5120
ROW_SHIFT = 8
GLOB_DUMMY = N_NODES
E_PAD = 163840
ROWS_PER_SUB = 80
ZERO_PER_SUB = 320
OUT_PER_SUB = NR // 16
HIST_ROWS = 79


def _sc_aggregate(x2, src2, dst2):
  mesh = plsc.VectorSubcoreMesh(core_axis_name="c", subcore_axis_name="s")

  @functools.partial(
      pl.kernel,
      mesh=mesh,
      compiler_params=pltpu.CompilerParams(needs_layout_passes=False),
      out_type=[
          jax.ShapeDtypeStruct((2, N_PASSES * NR, DH), jnp.float32),
          jax.ShapeDtypeStruct((16, HIST_ROWS, 128), jnp.float32),
      ],
      scratch_types=[
          pltpu.VMEM((ROWS_PER_SUB, 128), jnp.int32),
          pltpu.VMEM((ROWS_PER_SUB, 128), jnp.int32),
          pltpu.VMEM((ROWS_PER_SUB, 128), jnp.int32),
          pltpu.VMEM((128, DH), jnp.float32),
          pltpu.VMEM((128, DH), jnp.float32),
          pltpu.VMEM((HIST_ROWS, 128), jnp.float32),
          pltpu.VMEM((128, DH), jnp.float32),
          pltpu.VMEM_SHARED((ACC_ROWS, DH), jnp.float32),
          pltpu.SemaphoreType.DMA,
          pltpu.SemaphoreType.DMA,
      ],
  )
  def agg_kernel(x2_hbm, src_hbm, dst_hbm, out_hbm, cnt_hbm,
                 src_v, dst_v, dstp_v, rows_a, rows_b, hist_v, z_v, acc_sh,
                 sem_a, sem_b):
    rows_v = rows_a
    c = lax.axis_index("c")
    s = lax.axis_index("s")

    def fill_row(r, _):
      for k in range(DH // 16):
        z_v[r, pl.ds(k * 16, 16)] = jnp.zeros((16,), jnp.float32)
      return 0
    lax.fori_loop(0, 128, fill_row, 0)

    def hist_zero_row(r, _):
      for k in range(128 // 16):
        hist_v[r, pl.ds(k * 16, 16)] = jnp.zeros((16,), jnp.float32)
      return 0
    lax.fori_loop(0, HIST_ROWS, hist_zero_row, 0)

    row0 = s * ROWS_PER_SUB
    pltpu.sync_copy(src_hbm.at[pl.ds(row0, ROWS_PER_SUB)], src_v)
    pltpu.sync_copy(dst_hbm.at[pl.ds(row0, ROWS_PER_SUB)], dst_v)
    off = c * N_NODES

    def src_off_row(r, _):
      for k in range(128 // 16):
        src_v[r, pl.ds(k * 16, 16)] = src_v[r, pl.ds(k * 16, 16)] + off
      return 0
    lax.fori_loop(0, ROWS_PER_SUB, src_off_row, 0)

    @pl.when(c == 0)
    def _histogram():
      ones16 = jnp.ones((16,), jnp.float32)

      def hist_row(r, _):
        for k in range(128 // 16):
          v = dst_v[r, pl.ds(k * 16, 16)]
          plsc.addupdate_scatter(
              hist_v, [lax.shift_right_logical(v, 7), v & 127], ones16)
        return 0
      lax.fori_loop(0, ROWS_PER_SUB, hist_row, 0)
      pltpu.sync_copy(hist_v, cnt_hbm.at[s])

    for p in range(N_PASSES):
      cb = s * ZERO_PER_SUB
      pltpu.sync_copy(z_v, acc_sh.at[pl.ds(cb, 128)])
      pltpu.sync_copy(z_v, acc_sh.at[pl.ds(cb + 128, 128)])
      pltpu.sync_copy(z_v.at[pl.ds(0, 64)], acc_sh.at[pl.ds(cb + 256, 64)])

      dummy_p = 0 if p == 0 else (N_NODES + ROW_SHIFT - NR)
      def remap_row(r, _):
        for k in range(128 // 16):
          v = dst_v[r, pl.ds(k * 16, 16)]
          d = v + (ROW_SHIFT - p * NR)
          ok = (d >= 0) & (d < NR)
          dstp_v[r, pl.ds(k * 16, 16)] = jnp.where(
              ok, d, jnp.full((16,), dummy_p, jnp.int32))
        return 0
      lax.fori_loop(0, ROWS_PER_SUB, remap_row, 0)

      plsc.subcore_barrier()

      pltpu.async_copy(x2_hbm.at[src_v.at[0]], rows_a, sem_a)
      pltpu.async_copy(x2_hbm.at[src_v.at[1]], rows_b, sem_b)

      def edge_pair(t, _):
        j0 = 2 * t
        pltpu.make_async_copy(
            x2_hbm.at[src_v.at[j0]], rows_a, sem_a).wait()
        pltpu.sync_copy(rows_a, acc_sh.at[dstp_v.at[j0]], add=True)

        @pl.when(t < (ROWS_PER_SUB // 2) - 1)
        def _next_a():
          pltpu.async_copy(x2_hbm.at[src_v.at[j0 + 2]], rows_a, sem_a)

        pltpu.make_async_copy(
            x2_hbm.at[src_v.at[j0 + 1]], rows_b, sem_b).wait()
        pltpu.sync_copy(rows_b, acc_sh.at[dstp_v.at[j0 + 1]], add=True)

        @pl.when(t < (ROWS_PER_SUB // 2) - 1)
        def _next_b():
          pltpu.async_copy(x2_hbm.at[src_v.at[j0 + 3]], rows_b, sem_b)
        return 0
      lax.fori_loop(0, ROWS_PER_SUB // 2, edge_pair, 0)

      plsc.subcore_barrier()

      ob = s * OUT_PER_SUB
      for (o, w) in ((0, 128), (128, 128), (256, 64)):
        pltpu.sync_copy(acc_sh.at[pl.ds(ob + o, w)], rows_v.at[pl.ds(0, w)])
        pltpu.sync_copy(rows_v.at[pl.ds(0, w)],
                        out_hbm.at[c, pl.ds(p * NR + ob + o, w)])

      if p + 1 < N_PASSES:
        plsc.subcore_barrier()

  return agg_kernel(x2, src2, dst2)


def _tc_body(sumL_ref, sumR_ref, cnt_ref, x_ref, w_ref, b_ref, out_ref):
  cnt = jnp.sum(cnt_ref[...], axis=1)[:, None]
  r = 1.0 / jnp.maximum(cnt, 1.0)
  aggL = sumL_ref[...] * r
  aggR = sumR_ref[...] * r
  out_ref[...] = (
      jnp.dot(aggL, w_ref[0:DH, :], preferred_element_type=jnp.float32)
      + jnp.dot(aggR, w_ref[DH:D_IN, :], preferred_element_type=jnp.float32)
      + jnp.dot(x_ref[...], w_ref[D_IN:, :], preferred_element_type=jnp.float32)
      + b_ref[...]
  )


def _tc_matmul(sumL, sumR, cnt, x, w_all, b_all):
  blk = 1000
  grid = (N_NODES // blk,)
  return pl.pallas_call(
      _tc_body,
      grid=grid,
      in_specs=[
          pl.BlockSpec((blk, DH), lambda i: (i, 0)),
          pl.BlockSpec((blk, DH), lambda i: (i, 0)),
          pl.BlockSpec((blk, 16), lambda i: (i, 0)),
          pl.BlockSpec((blk, D_IN), lambda i: (i, 0)),
          pl.BlockSpec((2 * D_IN, 2 * D_OUT), lambda i: (0, 0)),
          pl.BlockSpec((1, 2 * D_OUT), lambda i: (0, 0)),
      ],
      out_specs=pl.BlockSpec((blk, 2 * D_OUT), lambda i: (i, 0)),
      out_shape=jax.ShapeDtypeStruct((N_NODES, 2 * D_OUT), jnp.float32),
  )(sumL, sumR, cnt, x, w_all, b_all)


def kernel(x, edge_index, Wl_mu, bl_mu, Wr_mu, Wl_ls, bl_ls, Wr_ls):
  x2 = jnp.concatenate([x[:, :DH], x[:, DH:]], axis=0)
  src = edge_index[0]
  dst = edge_index[1]
  pad = E_PAD - N_EDGES
  srcp = jnp.concatenate([src, jnp.zeros((pad,), jnp.int32)])
  dstp = jnp.concatenate([dst, jnp.full((pad,), GLOB_DUMMY, jnp.int32)])
  src2 = srcp.reshape(E_PAD // 128, 128)
  dst2 = dstp.reshape(E_PAD // 128, 128)

  summed2, hist = _sc_aggregate(x2, src2, dst2)
  summed2 = summed2[:, ROW_SHIFT:ROW_SHIFT + N_NODES, :]
  cnt16 = hist.reshape(16, HIST_ROWS * 128).T[:N_NODES]

  w_all = jnp.concatenate(
      [jnp.concatenate([Wl_mu.T, Wl_ls.T], axis=1),
       jnp.concatenate([Wr_mu.T, Wr_ls.T], axis=1)], axis=0)
  b_all = jnp.concatenate([bl_mu, bl_ls]).reshape(1, 2 * D_OUT)

  out = _tc_matmul(summed2[0], summed2[1], cnt16, x, w_all, b_all)
  return (out[:, :D_OUT], out[:, D_OUT:])


# speedup vs baseline: 2.2362x; 1.0092x over previous
import functools

import jax
import jax.numpy as jnp
from jax import lax
from jax.experimental import pallas as pl
from jax.experimental.pallas import tpu as pltpu
from jax.experimental.pallas import tpu_sc as plsc

N_NODES = 10000
N_EDGES = 160000
D_IN = 256
D_OUT = 256
DH = 128
N_PASSES = 2
NR = 5120
ACC_ROWS = 5120
ROW_SHIFT = 8
GLOB_DUMMY = N_NODES
DUMMY1 = N_NODES + ROW_SHIFT - NR
E_PAD = 163840
ROWS_PER_SUB = 80
ZERO_PER_SUB = 320
OUT_PER_SUB = NR // 16


def _sc_aggregate(x2, src2, dst2):
  mesh = plsc.VectorSubcoreMesh(core_axis_name="c", subcore_axis_name="s")

  @functools.partial(
      pl.kernel,
      mesh=mesh,
      compiler_params=pltpu.CompilerParams(needs_layout_passes=False),
      out_type=[
          jax.ShapeDtypeStruct((2, N_PASSES * NR, DH), jnp.float32),
              jax.ShapeDtypeStruct((16, ROWS_PER_SUB, 128), jnp.int32),
      ],
      scratch_types=[
          pltpu.VMEM((ROWS_PER_SUB, 128), jnp.int32),
          pltpu.VMEM((16, 128), jnp.int32),
          pltpu.VMEM((ROWS_PER_SUB, 128), jnp.int32),
          pltpu.VMEM((ROWS_PER_SUB, 128), jnp.int32),
          pltpu.VMEM((ROWS_PER_SUB, 128), jnp.int32),
          pltpu.VMEM((ROWS_PER_SUB, 128), jnp.int32),
          pltpu.VMEM((128, DH), jnp.float32),
          pltpu.VMEM((128, DH), jnp.float32),
          pltpu.VMEM_SHARED((ACC_ROWS, DH), jnp.float32),
          pltpu.SemaphoreType.DMA,
          pltpu.SemaphoreType.DMA,
      ],
  )
  def agg_kernel(x2_hbm, src_hbm, dst_hbm, out_hbm, cnt_hbm,
                 src_v, dstc, srcb0, dstb0, srcb1, dstb1,
                 rows_a, rows_b, acc_sh, sem_a, sem_b):
    rows_v = rows_a
    c = lax.axis_index("c")
    s = lax.axis_index("s")

    row0 = s * ROWS_PER_SUB
    pltpu.sync_copy(src_hbm.at[pl.ds(row0, ROWS_PER_SUB)], src_v)
    off = c * N_NODES

    def src_off_row(r, _):
      for k in range(128 // 16):
        src_v[r, pl.ds(k * 16, 16)] = src_v[r, pl.ds(k * 16, 16)] + off
      return 0
    lax.fori_loop(0, ROWS_PER_SUB, src_off_row, 0)

    @pl.when(c == 0)
    def _histogram():
      def hz_row(r, _):
        for k in range(128 // 16):
          dstb1[r, pl.ds(k * 16, 16)] = jnp.zeros((16,), jnp.int32)
        return 0
      lax.fori_loop(0, ROWS_PER_SUB, hz_row, 0)
      ones16 = jnp.ones((16,), jnp.int32)
      for q in range(ROWS_PER_SUB // 16):
        pltpu.sync_copy(dst_hbm.at[pl.ds(row0 + q * 16, 16)], dstc)

        def hist_row(r, _):
          for k in range(128 // 16):
            v = dstc[r, pl.ds(k * 16, 16)]
            plsc.addupdate_scatter(
                dstb1, [lax.shift_right_logical(v, 7), v & 127], ones16)
          return 0
        lax.fori_loop(0, 16, hist_row, 0)
      pltpu.sync_copy(dstb1, cnt_hbm.at[s])

    def bucket_fill_row(r, _):
      z16 = jnp.zeros((16,), jnp.int32)
      d16 = jnp.full((16,), DUMMY1, jnp.int32)
      for k in range(128 // 16):
        srcb0[r, pl.ds(k * 16, 16)] = z16
        srcb1[r, pl.ds(k * 16, 16)] = z16
        dstb0[r, pl.ds(k * 16, 16)] = z16
        dstb1[r, pl.ds(k * 16, 16)] = d16
      return 0
    lax.fori_loop(0, ROWS_PER_SUB, bucket_fill_row, 0)

    iota16 = lax.iota(jnp.int32, 16)

    def make_compact_row(q):
     def compact_row(r, carry):
      n0, n1 = carry
      for k in range(128 // 16):
        vd = dstc[r, pl.ds(k * 16, 16)]
        vs = src_v[q * 16 + r, pl.ds(k * 16, 16)]
        d0 = vd + ROW_SHIFT
        m0 = d0 < NR
        cum0 = plsc.cumsum(m0.astype(jnp.int32))
        pos0 = n0 + cum0 - 1
        plsc.store_scatter(
            srcb0, [lax.shift_right_logical(pos0, 7), pos0 & 127], vs,
            mask=m0)
        plsc.store_scatter(
            dstb0, [lax.shift_right_logical(pos0, 7), pos0 & 127], d0,
            mask=m0)
        m1 = jnp.logical_not(m0)
        pos1 = n1 + (iota16 + 1 - cum0) - 1
        plsc.store_scatter(
            srcb1, [lax.shift_right_logical(pos1, 7), pos1 & 127], vs,
            mask=m1)
        plsc.store_scatter(
            dstb1, [lax.shift_right_logical(pos1, 7), pos1 & 127], d0 - NR,
            mask=m1)
        t0 = jnp.sum(m0.astype(jnp.int32))
        n0 = n0 + t0
        n1 = n1 + (16 - t0)
      return (n0, n1)
     return compact_row

    carry = (jnp.int32(0), jnp.int32(0))
    for q in range(ROWS_PER_SUB // 16):
      pltpu.sync_copy(dst_hbm.at[pl.ds(row0 + q * 16, 16)], dstc)
      carry = lax.fori_loop(0, 16, make_compact_row(q), carry)
    n0, n1 = carry
    counts = (n0, n1)
    bsrc = (srcb0, srcb1)
    bdst = (dstb0, dstb1)

    for p in range(N_PASSES):
      def zfill_row(r, _):
        for k in range(DH // 16):
          rows_a[r, pl.ds(k * 16, 16)] = jnp.zeros((16,), jnp.float32)
        return 0
      lax.fori_loop(0, 128, zfill_row, 0)
      cb = s * ZERO_PER_SUB
      pltpu.sync_copy(rows_a, acc_sh.at[pl.ds(cb, 128)])
      pltpu.sync_copy(rows_a, acc_sh.at[pl.ds(cb + 128, 128)])
      pltpu.sync_copy(rows_a.at[pl.ds(0, 64)], acc_sh.at[pl.ds(cb + 256, 64)])

      plsc.subcore_barrier()

      sb, db, n_p = bsrc[p], bdst[p], counts[p]
      pairs = jnp.maximum(((n_p + 127) >> 7) + 1, 2) >> 1

      pltpu.async_copy(x2_hbm.at[sb.at[0]], rows_a, sem_a)
      pltpu.async_copy(x2_hbm.at[sb.at[1]], rows_b, sem_b)

      def edge_pair(t, _):
        j0 = 2 * t
        pltpu.make_async_copy(x2_hbm.at[sb.at[j0]], rows_a, sem_a).wait()
        pltpu.sync_copy(rows_a, acc_sh.at[db.at[j0]], add=True)

        @pl.when(t < pairs - 1)
        def _next_a():
          pltpu.async_copy(x2_hbm.at[sb.at[j0 + 2]], rows_a, sem_a)

        pltpu.make_async_copy(x2_hbm.at[sb.at[j0 + 1]], rows_b, sem_b).wait()
        pltpu.sync_copy(rows_b, acc_sh.at[db.at[j0 + 1]], add=True)

        @pl.when(t < pairs - 1)
        def _next_b():
          pltpu.async_copy(x2_hbm.at[sb.at[j0 + 3]], rows_b, sem_b)
        return 0
      lax.fori_loop(0, pairs, edge_pair, 0)

      plsc.subcore_barrier()

      ob = s * OUT_PER_SUB
      for (o, w) in ((0, 128), (128, 128), (256, 64)):
        pltpu.sync_copy(acc_sh.at[pl.ds(ob + o, w)], rows_v.at[pl.ds(0, w)])
        pltpu.sync_copy(rows_v.at[pl.ds(0, w)],
                        out_hbm.at[c, pl.ds(p * NR + ob + o, w)])

      if p + 1 < N_PASSES:
        plsc.subcore_barrier()

  return agg_kernel(x2, src2, dst2)


def _tc_body(sumL_ref, sumR_ref, cnt_ref, x_ref, w_ref, b_ref, out_ref):
  cnt = jnp.sum(cnt_ref[...], axis=1).astype(jnp.float32)[:, None]
  r = 1.0 / jnp.maximum(cnt, 1.0)
  aggL = sumL_ref[...] * r
  aggR = sumR_ref[...] * r
  out_ref[...] = (
      jnp.dot(aggL, w_ref[0:DH, :], preferred_element_type=jnp.float32)
      + jnp.dot(aggR, w_ref[DH:D_IN, :], preferred_element_type=jnp.float32)
      + jnp.dot(x_ref[...], w_ref[D_IN:, :], preferred_element_type=jnp.float32)
      + b_ref[...]
  )


def _tc_matmul(sumL, sumR, cnt, x, w_all, b_all):
  blk = 1000
  grid = (N_NODES // blk,)
  return pl.pallas_call(
      _tc_body,
      grid=grid,
      in_specs=[
          pl.BlockSpec((blk, DH), lambda i: (i, 0)),
          pl.BlockSpec((blk, DH), lambda i: (i, 0)),
          pl.BlockSpec((blk, 16), lambda i: (i, 0)),
          pl.BlockSpec((blk, D_IN), lambda i: (i, 0)),
          pl.BlockSpec((2 * D_IN, 2 * D_OUT), lambda i: (0, 0)),
          pl.BlockSpec((1, 2 * D_OUT), lambda i: (0, 0)),
      ],
      out_specs=pl.BlockSpec((blk, 2 * D_OUT), lambda i: (i, 0)),
      out_shape=jax.ShapeDtypeStruct((N_NODES, 2 * D_OUT), jnp.float32),
  )(sumL, sumR, cnt, x, w_all, b_all)


def kernel(x, edge_index, Wl_mu, bl_mu, Wr_mu, Wl_ls, bl_ls, Wr_ls):
  x2 = jnp.concatenate([x[:, :DH], x[:, DH:]], axis=0)
  src = edge_index[0]
  dst = edge_index[1]
  pad = E_PAD - N_EDGES
  srcp = jnp.concatenate([src, jnp.zeros((pad,), jnp.int32)])
  dstp = jnp.concatenate([dst, jnp.full((pad,), GLOB_DUMMY, jnp.int32)])
  src2 = srcp.reshape(E_PAD // 128, 128)
  dst2 = dstp.reshape(E_PAD // 128, 128)

  summed2, hist = _sc_aggregate(x2, src2, dst2)
  summed2 = summed2[:, ROW_SHIFT:ROW_SHIFT + N_NODES, :]
  cnt16 = hist.reshape(16, ROWS_PER_SUB * 128).T[:N_NODES]

  w_all = jnp.concatenate(
      [jnp.concatenate([Wl_mu.T, Wl_ls.T], axis=1),
       jnp.concatenate([Wr_mu.T, Wr_ls.T], axis=1)], axis=0)
  b_all = jnp.concatenate([bl_mu, bl_ls]).reshape(1, 2 * D_OUT)

  out = _tc_matmul(summed2[0], summed2[1], cnt16, x, w_all, b_all)
  return (out[:, :D_OUT], out[:, D_OUT:])


# speedup vs baseline: 2.2546x; 1.0082x over previous
import functools

import jax
import jax.numpy as jnp
from jax import lax
from jax.experimental import pallas as pl
from jax.experimental.pallas import tpu as pltpu
from jax.experimental.pallas import tpu_sc as plsc

N_NODES = 10000
N_EDGES = 160000
D_IN = 256
D_OUT = 256
DH = 128
N_PASSES = 2
NR = 5120
ACC_ROWS = 5120
ROW_SHIFT = 8
GLOB_DUMMY = N_NODES
DUMMY1 = N_NODES + ROW_SHIFT - NR
E_PAD = 163840
ROWS_PER_SUB = 80
ZERO_PER_SUB = 320
OUT_PER_SUB = NR // 16


def _sc_aggregate(x2, src2, dst2):
  mesh = plsc.VectorSubcoreMesh(core_axis_name="c", subcore_axis_name="s")

  @functools.partial(
      pl.kernel,
      mesh=mesh,
      compiler_params=pltpu.CompilerParams(needs_layout_passes=False),
      out_type=[
          jax.ShapeDtypeStruct((2, N_PASSES * NR, DH), jnp.float32),
              jax.ShapeDtypeStruct((16, ROWS_PER_SUB, 128), jnp.int32),
      ],
      scratch_types=[
          pltpu.VMEM((ROWS_PER_SUB, 128), jnp.int32),
          pltpu.VMEM((16, 128), jnp.int32),
          pltpu.VMEM((ROWS_PER_SUB, 128), jnp.int32),
          pltpu.VMEM((ROWS_PER_SUB, 128), jnp.int32),
          pltpu.VMEM((ROWS_PER_SUB, 128), jnp.int32),
          pltpu.VMEM((ROWS_PER_SUB, 128), jnp.int32),
          pltpu.VMEM((128, DH), jnp.float32),
          pltpu.VMEM((128, DH), jnp.float32),
          pltpu.VMEM_SHARED((ACC_ROWS, DH), jnp.float32),
          pltpu.SemaphoreType.DMA,
          pltpu.SemaphoreType.DMA,
          pltpu.SemaphoreType.DMA,
          pltpu.SemaphoreType.DMA,
      ],
  )
  def agg_kernel(x2_hbm, src_hbm, dst_hbm, out_hbm, cnt_hbm,
                 src_v, dstc, srcb0, dstb0, srcb1, dstb1,
                 rows_a, rows_b, acc_sh, sem_a0, sem_a1, sem_b0, sem_b1):
    rows_v = rows_a
    c = lax.axis_index("c")
    s = lax.axis_index("s")

    row0 = s * ROWS_PER_SUB
    pltpu.sync_copy(src_hbm.at[pl.ds(row0, ROWS_PER_SUB)], src_v)
    off = c * N_NODES

    def src_off_row(r, _):
      for k in range(128 // 16):
        src_v[r, pl.ds(k * 16, 16)] = src_v[r, pl.ds(k * 16, 16)] + off
      return 0
    lax.fori_loop(0, ROWS_PER_SUB, src_off_row, 0)

    @pl.when(c == 0)
    def _histogram():
      def hz_row(r, _):
        for k in range(128 // 16):
          dstb1[r, pl.ds(k * 16, 16)] = jnp.zeros((16,), jnp.int32)
        return 0
      lax.fori_loop(0, ROWS_PER_SUB, hz_row, 0)
      ones16 = jnp.ones((16,), jnp.int32)
      for q in range(ROWS_PER_SUB // 16):
        pltpu.sync_copy(dst_hbm.at[pl.ds(row0 + q * 16, 16)], dstc)

        def hist_row(r, _):
          for k in range(128 // 16):
            v = dstc[r, pl.ds(k * 16, 16)]
            plsc.addupdate_scatter(
                dstb1, [lax.shift_right_logical(v, 7), v & 127], ones16)
          return 0
        lax.fori_loop(0, 16, hist_row, 0)
      pltpu.sync_copy(dstb1, cnt_hbm.at[s])

    def bucket_fill_row(r, _):
      z16 = jnp.zeros((16,), jnp.int32)
      d16 = jnp.full((16,), DUMMY1, jnp.int32)
      for k in range(128 // 16):
        srcb0[r, pl.ds(k * 16, 16)] = z16
        srcb1[r, pl.ds(k * 16, 16)] = z16
        dstb0[r, pl.ds(k * 16, 16)] = z16
        dstb1[r, pl.ds(k * 16, 16)] = d16
      return 0
    lax.fori_loop(0, ROWS_PER_SUB, bucket_fill_row, 0)

    iota16 = lax.iota(jnp.int32, 16)

    def make_compact_row(q):
     def compact_row(r, carry):
      n0, n1 = carry
      for k in range(128 // 16):
        vd = dstc[r, pl.ds(k * 16, 16)]
        vs = src_v[q * 16 + r, pl.ds(k * 16, 16)]
        d0 = vd + ROW_SHIFT
        m0 = d0 < NR
        cum0 = plsc.cumsum(m0.astype(jnp.int32))
        pos0 = n0 + cum0 - 1
        plsc.store_scatter(
            srcb0, [lax.shift_right_logical(pos0, 7), pos0 & 127], vs,
            mask=m0)
        plsc.store_scatter(
            dstb0, [lax.shift_right_logical(pos0, 7), pos0 & 127], d0,
            mask=m0)
        m1 = jnp.logical_not(m0)
        pos1 = n1 + (iota16 + 1 - cum0) - 1
        plsc.store_scatter(
            srcb1, [lax.shift_right_logical(pos1, 7), pos1 & 127], vs,
            mask=m1)
        plsc.store_scatter(
            dstb1, [lax.shift_right_logical(pos1, 7), pos1 & 127], d0 - NR,
            mask=m1)
        t0 = jnp.sum(m0.astype(jnp.int32))
        n0 = n0 + t0
        n1 = n1 + (16 - t0)
      return (n0, n1)
     return compact_row

    carry = (jnp.int32(0), jnp.int32(0))
    for q in range(ROWS_PER_SUB // 16):
      pltpu.sync_copy(dst_hbm.at[pl.ds(row0 + q * 16, 16)], dstc)
      carry = lax.fori_loop(0, 16, make_compact_row(q), carry)
    n0, n1 = carry
    counts = (n0, n1)
    bsrc = (srcb0, srcb1)
    bdst = (dstb0, dstb1)

    for p in range(N_PASSES):
      def zfill_row(r, _):
        for k in range(DH // 16):
          rows_a[r, pl.ds(k * 16, 16)] = jnp.zeros((16,), jnp.float32)
        return 0
      lax.fori_loop(0, 128, zfill_row, 0)
      cb = s * ZERO_PER_SUB
      pltpu.sync_copy(rows_a, acc_sh.at[pl.ds(cb, 128)])
      pltpu.sync_copy(rows_a, acc_sh.at[pl.ds(cb + 128, 128)])
      pltpu.sync_copy(rows_a.at[pl.ds(0, 64)], acc_sh.at[pl.ds(cb + 256, 64)])

      plsc.subcore_barrier()

      sb, db, n_p = bsrc[p], bdst[p], counts[p]
      pairs = jnp.maximum(((n_p + 127) >> 7) + 1, 2) >> 1

      def gather_halves(j, buf, s0, s1):
        pltpu.async_copy(x2_hbm.at[sb.at[j, pl.ds(0, 64)]],
                         buf.at[pl.ds(0, 64)], s0)
        pltpu.async_copy(x2_hbm.at[sb.at[j, pl.ds(64, 64)]],
                         buf.at[pl.ds(64, 64)], s1)

      def wait_halves(j, buf, s0, s1):
        pltpu.make_async_copy(x2_hbm.at[sb.at[j, pl.ds(0, 64)]],
                              buf.at[pl.ds(0, 64)], s0).wait()
        pltpu.make_async_copy(x2_hbm.at[sb.at[j, pl.ds(64, 64)]],
                              buf.at[pl.ds(64, 64)], s1).wait()

      gather_halves(0, rows_a, sem_a0, sem_a1)
      gather_halves(1, rows_b, sem_b0, sem_b1)

      def edge_pair(t, _):
        j0 = 2 * t
        wait_halves(j0, rows_a, sem_a0, sem_a1)
        pltpu.sync_copy(rows_a, acc_sh.at[db.at[j0]], add=True)

        @pl.when(t < pairs - 1)
        def _next_a():
          gather_halves(j0 + 2, rows_a, sem_a0, sem_a1)

        wait_halves(j0 + 1, rows_b, sem_b0, sem_b1)
        pltpu.sync_copy(rows_b, acc_sh.at[db.at[j0 + 1]], add=True)

        @pl.when(t < pairs - 1)
        def _next_b():
          gather_halves(j0 + 3, rows_b, sem_b0, sem_b1)
        return 0
      lax.fori_loop(0, pairs, edge_pair, 0)

      plsc.subcore_barrier()

      ob = s * OUT_PER_SUB
      for (o, w) in ((0, 128), (128, 128), (256, 64)):
        pltpu.sync_copy(acc_sh.at[pl.ds(ob + o, w)], rows_v.at[pl.ds(0, w)])
        pltpu.sync_copy(rows_v.at[pl.ds(0, w)],
                        out_hbm.at[c, pl.ds(p * NR + ob + o, w)])

      if p + 1 < N_PASSES:
        plsc.subcore_barrier()

  return agg_kernel(x2, src2, dst2)


def _tc_body(sumL_ref, sumR_ref, cnt_ref, x_ref, w_ref, b_ref, out_ref):
  cnt = jnp.sum(cnt_ref[...], axis=1).astype(jnp.float32)[:, None]
  r = 1.0 / jnp.maximum(cnt, 1.0)
  aggL = sumL_ref[...] * r
  aggR = sumR_ref[...] * r
  out_ref[...] = (
      jnp.dot(aggL, w_ref[0:DH, :], preferred_element_type=jnp.float32)
      + jnp.dot(aggR, w_ref[DH:D_IN, :], preferred_element_type=jnp.float32)
      + jnp.dot(x_ref[...], w_ref[D_IN:, :], preferred_element_type=jnp.float32)
      + b_ref[...]
  )


def _tc_matmul(sumL, sumR, cnt, x, w_all, b_all):
  blk = 1000
  grid = (N_NODES // blk,)
  return pl.pallas_call(
      _tc_body,
      grid=grid,
      in_specs=[
          pl.BlockSpec((blk, DH), lambda i: (i, 0)),
          pl.BlockSpec((blk, DH), lambda i: (i, 0)),
          pl.BlockSpec((blk, 16), lambda i: (i, 0)),
          pl.BlockSpec((blk, D_IN), lambda i: (i, 0)),
          pl.BlockSpec((2 * D_IN, 2 * D_OUT), lambda i: (0, 0)),
          pl.BlockSpec((1, 2 * D_OUT), lambda i: (0, 0)),
      ],
      out_specs=pl.BlockSpec((blk, 2 * D_OUT), lambda i: (i, 0)),
      out_shape=jax.ShapeDtypeStruct((N_NODES, 2 * D_OUT), jnp.float32),
  )(sumL, sumR, cnt, x, w_all, b_all)


def kernel(x, edge_index, Wl_mu, bl_mu, Wr_mu, Wl_ls, bl_ls, Wr_ls):
  x2 = jnp.concatenate([x[:, :DH], x[:, DH:]], axis=0)
  src = edge_index[0]
  dst = edge_index[1]
  pad = E_PAD - N_EDGES
  srcp = jnp.concatenate([src, jnp.zeros((pad,), jnp.int32)])
  dstp = jnp.concatenate([dst, jnp.full((pad,), GLOB_DUMMY, jnp.int32)])
  src2 = srcp.reshape(E_PAD // 128, 128)
  dst2 = dstp.reshape(E_PAD // 128, 128)

  summed2, hist = _sc_aggregate(x2, src2, dst2)
  summed2 = summed2[:, ROW_SHIFT:ROW_SHIFT + N_NODES, :]
  cnt16 = hist.reshape(16, ROWS_PER_SUB * 128).T[:N_NODES]

  w_all = jnp.concatenate(
      [jnp.concatenate([Wl_mu.T, Wl_ls.T], axis=1),
       jnp.concatenate([Wr_mu.T, Wr_ls.T], axis=1)], axis=0)
  b_all = jnp.concatenate([bl_mu, bl_ls]).reshape(1, 2 * D_OUT)

  out = _tc_matmul(summed2[0], summed2[1], cnt16, x, w_all, b_all)
  return (out[:, :D_OUT], out[:, D_OUT:])


# speedup vs baseline: 2.2560x; 1.0006x over previous
import functools

import jax
import jax.numpy as jnp
from jax import lax
from jax.experimental import pallas as pl
from jax.experimental.pallas import tpu as pltpu
from jax.experimental.pallas import tpu_sc as plsc

N_NODES = 10000
N_EDGES = 160000
D_IN = 256
D_OUT = 256
DH = 128
N_PASSES = 2
NR = 5120
ACC_ROWS = 5120
ROW_SHIFT = 8
GLOB_DUMMY = N_NODES
DUMMY1 = N_NODES + ROW_SHIFT - NR
E_PAD = 163840
ROWS_PER_SUB = 80
ZERO_PER_SUB = 320
OUT_PER_SUB = NR // 16


def _sc_aggregate(x2, src2, dst2):
  mesh = plsc.VectorSubcoreMesh(core_axis_name="c", subcore_axis_name="s")

  @functools.partial(
      pl.kernel,
      mesh=mesh,
      compiler_params=pltpu.CompilerParams(needs_layout_passes=False),
      out_type=[
          jax.ShapeDtypeStruct((2, N_PASSES * NR, DH), jnp.float32),
              jax.ShapeDtypeStruct((16, ROWS_PER_SUB, 128), jnp.int32),
      ],
      scratch_types=[
          pltpu.VMEM((ROWS_PER_SUB, 128), jnp.int32),
          pltpu.VMEM((16, 128), jnp.int32),
          pltpu.VMEM((ROWS_PER_SUB, 128), jnp.int32),
          pltpu.VMEM((ROWS_PER_SUB, 128), jnp.int32),
          pltpu.VMEM((ROWS_PER_SUB, 128), jnp.int32),
          pltpu.VMEM((ROWS_PER_SUB, 128), jnp.int32),
          pltpu.VMEM((128, DH), jnp.float32),
          pltpu.VMEM((128, DH), jnp.float32),
          pltpu.VMEM_SHARED((ACC_ROWS, DH), jnp.float32),
          pltpu.SemaphoreType.DMA,
          pltpu.SemaphoreType.DMA,
      ],
  )
  def agg_kernel(x2_hbm, src_hbm, dst_hbm, out_hbm, cnt_hbm,
                 src_v, dstc, srcb0, dstb0, srcb1, dstb1,
                 rows_a, rows_b, acc_sh, sem_a, sem_b):
    rows_v = rows_a
    c = lax.axis_index("c")
    s = lax.axis_index("s")

    row0 = s * ROWS_PER_SUB
    pltpu.sync_copy(src_hbm.at[pl.ds(row0, ROWS_PER_SUB)], src_v)
    off = c * N_NODES

    def src_off_row(r, _):
      for k in range(128 // 16):
        src_v[r, pl.ds(k * 16, 16)] = src_v[r, pl.ds(k * 16, 16)] + off
      return 0
    lax.fori_loop(0, ROWS_PER_SUB, src_off_row, 0)

    @pl.when(c == 0)
    def _histogram():
      def hz_row(r, _):
        for k in range(128 // 16):
          dstb1[r, pl.ds(k * 16, 16)] = jnp.zeros((16,), jnp.int32)
        return 0
      lax.fori_loop(0, ROWS_PER_SUB, hz_row, 0)
      ones16 = jnp.ones((16,), jnp.int32)
      for q in range(ROWS_PER_SUB // 16):
        pltpu.sync_copy(dst_hbm.at[pl.ds(row0 + q * 16, 16)], dstc)

        def hist_row(r, _):
          for k in range(128 // 16):
            v = dstc[r, pl.ds(k * 16, 16)]
            plsc.addupdate_scatter(
                dstb1, [lax.shift_right_logical(v, 7), v & 127], ones16)
          return 0
        lax.fori_loop(0, 16, hist_row, 0)
      pltpu.sync_copy(dstb1, cnt_hbm.at[s])

    def bucket_fill_row(r, _):
      z16 = jnp.zeros((16,), jnp.int32)
      d16 = jnp.full((16,), DUMMY1, jnp.int32)
      for k in range(128 // 16):
        srcb0[r, pl.ds(k * 16, 16)] = z16
        srcb1[r, pl.ds(k * 16, 16)] = z16
        dstb0[r, pl.ds(k * 16, 16)] = z16
        dstb1[r, pl.ds(k * 16, 16)] = d16
      return 0
    lax.fori_loop(0, ROWS_PER_SUB, bucket_fill_row, 0)

    iota16 = lax.iota(jnp.int32, 16)

    def make_compact_row(q):
     def compact_row(r, carry):
      n0, n1 = carry
      for k in range(128 // 16):
        vd = dstc[r, pl.ds(k * 16, 16)]
        vs = src_v[q * 16 + r, pl.ds(k * 16, 16)]
        d0 = vd + ROW_SHIFT
        m0 = d0 < NR
        cum0 = plsc.cumsum(m0.astype(jnp.int32))
        pos0 = n0 + cum0 - 1
        plsc.store_scatter(
            srcb0, [lax.shift_right_logical(pos0, 7), pos0 & 127], vs,
            mask=m0)
        plsc.store_scatter(
            dstb0, [lax.shift_right_logical(pos0, 7), pos0 & 127], d0,
            mask=m0)
        m1 = jnp.logical_not(m0)
        pos1 = n1 + (iota16 + 1 - cum0) - 1
        plsc.store_scatter(
            srcb1, [lax.shift_right_logical(pos1, 7), pos1 & 127], vs,
            mask=m1)
        plsc.store_scatter(
            dstb1, [lax.shift_right_logical(pos1, 7), pos1 & 127], d0 - NR,
            mask=m1)
        t0 = jnp.sum(m0.astype(jnp.int32))
        n0 = n0 + t0
        n1 = n1 + (16 - t0)
      return (n0, n1)
     return compact_row

    carry = (jnp.int32(0), jnp.int32(0))
    for q in range(ROWS_PER_SUB // 16):
      pltpu.sync_copy(dst_hbm.at[pl.ds(row0 + q * 16, 16)], dstc)
      carry = lax.fori_loop(0, 16, make_compact_row(q), carry)
    n0, n1 = carry
    counts = (n0, n1)
    bsrc = (srcb0, srcb1)
    bdst = (dstb0, dstb1)

    for p in range(N_PASSES):
      def zfill_row(r, _):
        for k in range(DH // 16):
          rows_a[r, pl.ds(k * 16, 16)] = jnp.zeros((16,), jnp.float32)
        return 0
      lax.fori_loop(0, 128, zfill_row, 0)
      cb = s * ZERO_PER_SUB
      pltpu.sync_copy(rows_a, acc_sh.at[pl.ds(cb, 128)])
      pltpu.sync_copy(rows_a, acc_sh.at[pl.ds(cb + 128, 128)])
      pltpu.sync_copy(rows_a.at[pl.ds(0, 64)], acc_sh.at[pl.ds(cb + 256, 64)])

      plsc.subcore_barrier()

      sb, db, n_p = bsrc[p], bdst[p], counts[p]
      pairs = jnp.maximum(((n_p + 127) >> 7) + 1, 2) >> 1

      pltpu.async_copy(x2_hbm.at[sb.at[0]], rows_a, sem_a)
      pltpu.async_copy(x2_hbm.at[sb.at[1]], rows_b, sem_b)

      def edge_pair(t, _):
        j0 = 2 * t
        pltpu.make_async_copy(x2_hbm.at[sb.at[j0]], rows_a, sem_a).wait()
        pltpu.sync_copy(rows_a, acc_sh.at[db.at[j0]], add=True)

        @pl.when(t < pairs - 1)
        def _next_a():
          pltpu.async_copy(x2_hbm.at[sb.at[j0 + 2]], rows_a, sem_a)

        pltpu.make_async_copy(x2_hbm.at[sb.at[j0 + 1]], rows_b, sem_b).wait()
        pltpu.sync_copy(rows_b, acc_sh.at[db.at[j0 + 1]], add=True)

        @pl.when(t < pairs - 1)
        def _next_b():
          pltpu.async_copy(x2_hbm.at[sb.at[j0 + 3]], rows_b, sem_b)
        return 0
      lax.fori_loop(0, pairs, edge_pair, 0)

      plsc.subcore_barrier()

      ob = s * OUT_PER_SUB
      for (o, w) in ((0, 128), (128, 128), (256, 64)):
        pltpu.sync_copy(acc_sh.at[pl.ds(ob + o, w)],
                        out_hbm.at[c, pl.ds(p * NR + ob + o, w)])

      if p + 1 < N_PASSES:
        plsc.subcore_barrier()

  return agg_kernel(x2, src2, dst2)


def _tc_body(sumL_ref, sumR_ref, cnt_ref, x_ref, w_ref, b_ref, out_ref):
  cnt = jnp.sum(cnt_ref[...], axis=1).astype(jnp.float32)[:, None]
  r = 1.0 / jnp.maximum(cnt, 1.0)
  aggL = sumL_ref[...] * r
  aggR = sumR_ref[...] * r
  out_ref[...] = (
      jnp.dot(aggL, w_ref[0:DH, :], preferred_element_type=jnp.float32)
      + jnp.dot(aggR, w_ref[DH:D_IN, :], preferred_element_type=jnp.float32)
      + jnp.dot(x_ref[...], w_ref[D_IN:, :], preferred_element_type=jnp.float32)
      + b_ref[...]
  )


def _tc_matmul(sumL, sumR, cnt, x, w_all, b_all):
  blk = 1000
  grid = (N_NODES // blk,)
  return pl.pallas_call(
      _tc_body,
      grid=grid,
      in_specs=[
          pl.BlockSpec((blk, DH), lambda i: (i, 0)),
          pl.BlockSpec((blk, DH), lambda i: (i, 0)),
          pl.BlockSpec((blk, 16), lambda i: (i, 0)),
          pl.BlockSpec((blk, D_IN), lambda i: (i, 0)),
          pl.BlockSpec((2 * D_IN, 2 * D_OUT), lambda i: (0, 0)),
          pl.BlockSpec((1, 2 * D_OUT), lambda i: (0, 0)),
      ],
      out_specs=pl.BlockSpec((blk, 2 * D_OUT), lambda i: (i, 0)),
      out_shape=jax.ShapeDtypeStruct((N_NODES, 2 * D_OUT), jnp.float32),
  )(sumL, sumR, cnt, x, w_all, b_all)


def kernel(x, edge_index, Wl_mu, bl_mu, Wr_mu, Wl_ls, bl_ls, Wr_ls):
  x2 = jnp.concatenate([x[:, :DH], x[:, DH:]], axis=0)
  src = edge_index[0]
  dst = edge_index[1]
  pad = E_PAD - N_EDGES
  srcp = jnp.concatenate([src, jnp.zeros((pad,), jnp.int32)])
  dstp = jnp.concatenate([dst, jnp.full((pad,), GLOB_DUMMY, jnp.int32)])
  src2 = srcp.reshape(E_PAD // 128, 128)
  dst2 = dstp.reshape(E_PAD // 128, 128)

  summed2, hist = _sc_aggregate(x2, src2, dst2)
  summed2 = summed2[:, ROW_SHIFT:ROW_SHIFT + N_NODES, :]
  cnt16 = hist.reshape(16, ROWS_PER_SUB * 128).T[:N_NODES]

  w_all = jnp.concatenate(
      [jnp.concatenate([Wl_mu.T, Wl_ls.T], axis=1),
       jnp.concatenate([Wr_mu.T, Wr_ls.T], axis=1)], axis=0)
  b_all = jnp.concatenate([bl_mu, bl_ls]).reshape(1, 2 * D_OUT)

  out = _tc_matmul(summed2[0], summed2[1], cnt16, x, w_all, b_all)
  return (out[:, :D_OUT], out[:, D_OUT:])


# speedup vs baseline: 2.2560x; 1.0000x over previous
import functools

import jax
import jax.numpy as jnp
from jax import lax
from jax.experimental import pallas as pl
from jax.experimental.pallas import tpu as pltpu
from jax.experimental.pallas import tpu_sc as plsc

N_NODES = 10000
N_EDGES = 160000
D_IN = 256
D_OUT = 256
DH = 128
N_PASSES = 2
NR = 5120
ACC_ROWS = 5120
ROW_SHIFT = 8
GLOB_DUMMY = N_NODES
DUMMY1 = N_NODES + ROW_SHIFT - NR
E_PAD = 163840
ROWS_PER_SUB = 80
ZERO_PER_SUB = 320
OUT_PER_SUB = NR // 16


def _sc_aggregate(x2, src2, dst2):
  mesh = plsc.VectorSubcoreMesh(core_axis_name="c", subcore_axis_name="s")

  @functools.partial(
      pl.kernel,
      mesh=mesh,
      compiler_params=pltpu.CompilerParams(needs_layout_passes=False),
      out_type=[
          jax.ShapeDtypeStruct((2, N_PASSES * NR, DH), jnp.float32),
              jax.ShapeDtypeStruct((16, ROWS_PER_SUB, 128), jnp.int32),
      ],
      scratch_types=[
          pltpu.VMEM((ROWS_PER_SUB, 128), jnp.int32),
          pltpu.VMEM((16, 128), jnp.int32),
          pltpu.VMEM((ROWS_PER_SUB, 128), jnp.int32),
          pltpu.VMEM((ROWS_PER_SUB, 128), jnp.int32),
          pltpu.VMEM((ROWS_PER_SUB, 128), jnp.int32),
          pltpu.VMEM((ROWS_PER_SUB, 128), jnp.int32),
          pltpu.VMEM((128, DH), jnp.float32),
          pltpu.VMEM((128, DH), jnp.float32),
          pltpu.VMEM((16, DH), jnp.float32),
          pltpu.VMEM_SHARED((ACC_ROWS, DH), jnp.float32),
          pltpu.SemaphoreType.DMA,
          pltpu.SemaphoreType.DMA,
      ],
  )
  def agg_kernel(x2_hbm, src_hbm, dst_hbm, out_hbm, cnt_hbm,
                 src_v, dstc, srcb0, dstb0, srcb1, dstb1,
                 rows_a, rows_b, z16_v, acc_sh, sem_a, sem_b):
    c = lax.axis_index("c")
    s = lax.axis_index("s")

    row0 = s * ROWS_PER_SUB
    pltpu.sync_copy(src_hbm.at[pl.ds(row0, ROWS_PER_SUB)], src_v)
    off = c * N_NODES

    def src_off_row(r, _):
      for k in range(128 // 16):
        src_v[r, pl.ds(k * 16, 16)] = src_v[r, pl.ds(k * 16, 16)] + off
      return 0
    lax.fori_loop(0, ROWS_PER_SUB, src_off_row, 0)

    @pl.when(c == 0)
    def _histogram():
      def hz_row(r, _):
        for k in range(128 // 16):
          dstb1[r, pl.ds(k * 16, 16)] = jnp.zeros((16,), jnp.int32)
        return 0
      lax.fori_loop(0, ROWS_PER_SUB, hz_row, 0)
      ones16 = jnp.ones((16,), jnp.int32)
      for q in range(ROWS_PER_SUB // 16):
        pltpu.sync_copy(dst_hbm.at[pl.ds(row0 + q * 16, 16)], dstc)

        def hist_row(r, _):
          for k in range(128 // 16):
            v = dstc[r, pl.ds(k * 16, 16)]
            plsc.addupdate_scatter(
                dstb1, [lax.shift_right_logical(v, 7), v & 127], ones16)
          return 0
        lax.fori_loop(0, 16, hist_row, 0)
      pltpu.sync_copy(dstb1, cnt_hbm.at[s])

    def bucket_fill_row(r, _):
      z16 = jnp.zeros((16,), jnp.int32)
      d16 = jnp.full((16,), DUMMY1, jnp.int32)
      for k in range(128 // 16):
        srcb0[r, pl.ds(k * 16, 16)] = z16
        srcb1[r, pl.ds(k * 16, 16)] = z16
        dstb0[r, pl.ds(k * 16, 16)] = z16
        dstb1[r, pl.ds(k * 16, 16)] = d16
      return 0
    lax.fori_loop(0, ROWS_PER_SUB, bucket_fill_row, 0)

    iota16 = lax.iota(jnp.int32, 16)

    def make_compact_row(q):
     def compact_row(r, carry):
      n0, n1 = carry
      for k in range(128 // 16):
        vd = dstc[r, pl.ds(k * 16, 16)]
        vs = src_v[q * 16 + r, pl.ds(k * 16, 16)]
        d0 = vd + ROW_SHIFT
        m0 = d0 < NR
        cum0 = plsc.cumsum(m0.astype(jnp.int32))
        pos0 = n0 + cum0 - 1
        plsc.store_scatter(
            srcb0, [lax.shift_right_logical(pos0, 7), pos0 & 127], vs,
            mask=m0)
        plsc.store_scatter(
            dstb0, [lax.shift_right_logical(pos0, 7), pos0 & 127], d0,
            mask=m0)
        m1 = jnp.logical_not(m0)
        pos1 = n1 + (iota16 + 1 - cum0) - 1
        plsc.store_scatter(
            srcb1, [lax.shift_right_logical(pos1, 7), pos1 & 127], vs,
            mask=m1)
        plsc.store_scatter(
            dstb1, [lax.shift_right_logical(pos1, 7), pos1 & 127], d0 - NR,
            mask=m1)
        t0 = jnp.sum(m0.astype(jnp.int32))
        n0 = n0 + t0
        n1 = n1 + (16 - t0)
      return (n0, n1)
     return compact_row

    carry = (jnp.int32(0), jnp.int32(0))
    for q in range(ROWS_PER_SUB // 16):
      pltpu.sync_copy(dst_hbm.at[pl.ds(row0 + q * 16, 16)], dstc)
      carry = lax.fori_loop(0, 16, make_compact_row(q), carry)
    n0, n1 = carry
    counts = (n0, n1)
    bsrc = (srcb0, srcb1)
    bdst = (dstb0, dstb1)

    def z16_row(r, _):
      for k in range(DH // 16):
        z16_v[r, pl.ds(k * 16, 16)] = jnp.zeros((16,), jnp.float32)
      return 0
    lax.fori_loop(0, 16, z16_row, 0)

    pltpu.async_copy(x2_hbm.at[srcb0.at[0]], rows_a, sem_a)
    pltpu.async_copy(x2_hbm.at[srcb0.at[1]], rows_b, sem_b)

    for p in range(N_PASSES):
      cb = s * ZERO_PER_SUB
      for zk in range(ZERO_PER_SUB // 16):
        pltpu.sync_copy(z16_v, acc_sh.at[pl.ds(cb + 16 * zk, 16)])

      plsc.subcore_barrier()

      sb, db, n_p = bsrc[p], bdst[p], counts[p]
      pairs = jnp.maximum(((n_p + 127) >> 7) + 1, 2) >> 1

      def edge_pair(t, _):
        j0 = 2 * t
        pltpu.make_async_copy(x2_hbm.at[sb.at[j0]], rows_a, sem_a).wait()
        pltpu.sync_copy(rows_a, acc_sh.at[db.at[j0]], add=True)

        @pl.when(t < pairs - 1)
        def _next_a():
          pltpu.async_copy(x2_hbm.at[sb.at[j0 + 2]], rows_a, sem_a)

        pltpu.make_async_copy(x2_hbm.at[sb.at[j0 + 1]], rows_b, sem_b).wait()
        pltpu.sync_copy(rows_b, acc_sh.at[db.at[j0 + 1]], add=True)

        @pl.when(t < pairs - 1)
        def _next_b():
          pltpu.async_copy(x2_hbm.at[sb.at[j0 + 3]], rows_b, sem_b)
        return 0
      lax.fori_loop(0, pairs, edge_pair, 0)

      if p + 1 < N_PASSES:
        pltpu.async_copy(x2_hbm.at[bsrc[p + 1].at[0]], rows_a, sem_a)
        pltpu.async_copy(x2_hbm.at[bsrc[p + 1].at[1]], rows_b, sem_b)

      plsc.subcore_barrier()

      ob = s * OUT_PER_SUB
      for (o, w) in ((0, 128), (128, 128), (256, 64)):
        pltpu.sync_copy(acc_sh.at[pl.ds(ob + o, w)],
                        out_hbm.at[c, pl.ds(p * NR + ob + o, w)])

      if p + 1 < N_PASSES:
        plsc.subcore_barrier()

  return agg_kernel(x2, src2, dst2)


def _tc_body(sumL_ref, sumR_ref, cnt_ref, x_ref, w_ref, b_ref, out_ref):
  cnt = jnp.sum(cnt_ref[...], axis=1).astype(jnp.float32)[:, None]
  r = 1.0 / jnp.maximum(cnt, 1.0)
  aggL = sumL_ref[...] * r
  aggR = sumR_ref[...] * r
  out_ref[...] = (
      jnp.dot(aggL, w_ref[0:DH, :], preferred_element_type=jnp.float32)
      + jnp.dot(aggR, w_ref[DH:D_IN, :], preferred_element_type=jnp.float32)
      + jnp.dot(x_ref[...], w_ref[D_IN:, :], preferred_element_type=jnp.float32)
      + b_ref[...]
  )


def _tc_matmul(sumL, sumR, cnt, x, w_all, b_all):
  blk = 1000
  grid = (N_NODES // blk,)
  return pl.pallas_call(
      _tc_body,
      grid=grid,
      in_specs=[
          pl.BlockSpec((blk, DH), lambda i: (i, 0)),
          pl.BlockSpec((blk, DH), lambda i: (i, 0)),
          pl.BlockSpec((blk, 16), lambda i: (i, 0)),
          pl.BlockSpec((blk, D_IN), lambda i: (i, 0)),
          pl.BlockSpec((2 * D_IN, 2 * D_OUT), lambda i: (0, 0)),
          pl.BlockSpec((1, 2 * D_OUT), lambda i: (0, 0)),
      ],
      out_specs=pl.BlockSpec((blk, 2 * D_OUT), lambda i: (i, 0)),
      out_shape=jax.ShapeDtypeStruct((N_NODES, 2 * D_OUT), jnp.float32),
  )(sumL, sumR, cnt, x, w_all, b_all)


def kernel(x, edge_index, Wl_mu, bl_mu, Wr_mu, Wl_ls, bl_ls, Wr_ls):
  x2 = jnp.concatenate([x[:, :DH], x[:, DH:]], axis=0)
  src = edge_index[0]
  dst = edge_index[1]
  pad = E_PAD - N_EDGES
  srcp = jnp.concatenate([src, jnp.zeros((pad,), jnp.int32)])
  dstp = jnp.concatenate([dst, jnp.full((pad,), GLOB_DUMMY, jnp.int32)])
  src2 = srcp.reshape(E_PAD // 128, 128)
  dst2 = dstp.reshape(E_PAD // 128, 128)

  summed2, hist = _sc_aggregate(x2, src2, dst2)
  summed2 = summed2[:, ROW_SHIFT:ROW_SHIFT + N_NODES, :]
  cnt16 = hist.reshape(16, ROWS_PER_SUB * 128).T[:N_NODES]

  w_all = jnp.concatenate(
      [jnp.concatenate([Wl_mu.T, Wl_ls.T], axis=1),
       jnp.concatenate([Wr_mu.T, Wr_ls.T], axis=1)], axis=0)
  b_all = jnp.concatenate([bl_mu, bl_ls]).reshape(1, 2 * D_OUT)

  out = _tc_matmul(summed2[0], summed2[1], cnt16, x, w_all, b_all)
  return (out[:, :D_OUT], out[:, D_OUT:])
